# baseline (speedup 1.0000x reference)
import functools

import jax
import jax.numpy as jnp
from jax import lax
from jax.experimental import pallas as pl
from jax.experimental.pallas import tpu as pltpu
from jax.experimental.pallas import tpu_sc as plsc

D_MODEL = 128
N_SITES = 512
ROW_LEN = 3 * N_SITES
L = 16
CHUNK = 128
NCHUNK = N_SITES // CHUNK


def _make_sc_kernel(batch: int):
    info = plsc.get_sparse_core_info()
    nc, ns = info.num_cores, info.num_subcores
    nw = nc * ns
    assert batch % nw == 0
    rows_per_w = batch // nw

    mesh = plsc.VectorSubcoreMesh(core_axis_name="c", subcore_axis_name="s")

    @functools.partial(
        pl.kernel,
        mesh=mesh,
        out_type=jax.ShapeDtypeStruct((batch * N_SITES, D_MODEL), jnp.float32),
        scratch_types=[
            pltpu.VMEM((ROW_LEN,), jnp.int32),
            pltpu.VMEM((NCHUNK, CHUNK), jnp.int32),
            pltpu.VMEM((NCHUNK, CHUNK, D_MODEL), jnp.float32),
            pltpu.SemaphoreType.DMA,
        ]
        + [pltpu.SemaphoreType.DMA for _ in range(NCHUNK)]
        + [pltpu.SemaphoreType.DMA for _ in range(NCHUNK)],
    )
    def body(n_hbm, emb_hbm, out_hbm, nrow_v, tok_v, rows_v, nsem, *sems):
        gsem = sems[:NCHUNK]
        wsem = sems[NCHUNK:]
        wid = lax.axis_index("s") * nc + lax.axis_index("c")
        row0 = wid * rows_per_w

        def stage_and_pack(row):
            pltpu.async_copy(
                n_hbm.at[pl.ds(row * ROW_LEN, ROW_LEN)], nrow_v, nsem
            ).wait()
            for c in range(NCHUNK):
                for j in range(CHUNK // L):
                    o = c * CHUNK + j * L
                    down = nrow_v[pl.ds(o, L)]
                    up = nrow_v[pl.ds(N_SITES + o, L)]
                    sp = nrow_v[pl.ds(2 * N_SITES + o, L)]
                    tok_v[c, pl.ds(j * L, L)] = (
                        up + 2 * down + 4 * ((sp + 1) >> 1)
                    )

        stage_and_pack(row0)
        for c in range(NCHUNK):
            pltpu.async_copy(emb_hbm.at[tok_v.at[c]], rows_v.at[c], gsem[c])

        def round_body(r, carry):
            row = row0 + r
            for c in range(NCHUNK):
                pltpu.make_async_copy(
                    emb_hbm.at[tok_v.at[c]], rows_v.at[c], gsem[c]
                ).wait()
                pltpu.async_copy(
                    rows_v.at[c],
                    out_hbm.at[pl.ds(row * N_SITES + c * CHUNK, CHUNK)],
                    wsem[c],
                )

            @pl.when(r + 1 < rows_per_w)
            def _():
                stage_and_pack(row + 1)
                for c in range(NCHUNK):
                    pltpu.make_async_copy(
                        rows_v.at[c],
                        out_hbm.at[pl.ds(row * N_SITES + c * CHUNK, CHUNK)],
                        wsem[c],
                    ).wait()
                    pltpu.async_copy(
                        emb_hbm.at[tok_v.at[c]], rows_v.at[c], gsem[c]
                    )

            return carry

        lax.fori_loop(0, rows_per_w, round_body, 0)

        last = row0 + rows_per_w - 1
        for c in range(NCHUNK):
            pltpu.make_async_copy(
                rows_v.at[c],
                out_hbm.at[pl.ds(last * N_SITES + c * CHUNK, CHUNK)],
                wsem[c],
            ).wait()

    return body


def kernel(n_flat, embedding):
    n = jnp.asarray(n_flat)
    if n.ndim == 1:
        n = n[None, :]
    batch = n.shape[0]
    body = _make_sc_kernel(batch)
    out = body(n.reshape(-1), embedding)
    return out.reshape(batch, N_SITES, D_MODEL)


# speedup vs baseline: 20.1208x; 20.1208x over previous
import functools

import jax
import jax.numpy as jnp
from jax import lax
from jax.experimental import pallas as pl
from jax.experimental.pallas import tpu as pltpu
from jax.experimental.pallas import tpu_sc as plsc

D_MODEL = 128
N_SITES = 512
ROW_LEN = 3 * N_SITES
L = 16
CHUNK = 128
NCHUNK = N_SITES // CHUNK


def _make_sc_kernel(batch: int):
    info = plsc.get_sparse_core_info()
    nc, ns = info.num_cores, info.num_subcores
    nw = nc * ns
    assert batch % nw == 0
    rows_per_w = batch // nw

    mesh = plsc.VectorSubcoreMesh(core_axis_name="c", subcore_axis_name="s")

    @functools.partial(
        pl.kernel,
        mesh=mesh,
        out_type=jax.ShapeDtypeStruct((batch * N_SITES, D_MODEL), jnp.float32),
        scratch_types=[
            pltpu.VMEM((ROW_LEN,), jnp.int32),
            pltpu.VMEM((NCHUNK, CHUNK), jnp.int32),
            pltpu.VMEM((NCHUNK, CHUNK, D_MODEL), jnp.float32),
            pltpu.VMEM_SHARED((8, D_MODEL), jnp.float32),
            pltpu.SemaphoreType.DMA,
        ]
        + [pltpu.SemaphoreType.DMA for _ in range(NCHUNK)]
        + [pltpu.SemaphoreType.DMA for _ in range(NCHUNK)],
    )
    def body(n_hbm, emb_hbm, out_hbm, nrow_v, tok_v, rows_v, emb_v, nsem, *sems):
        gsem = sems[:NCHUNK]
        wsem = sems[NCHUNK:]
        wid = lax.axis_index("s") * nc + lax.axis_index("c")
        row0 = wid * rows_per_w

        @pl.when(lax.axis_index("s") == 0)
        def _():
            pltpu.sync_copy(emb_hbm, emb_v)

        plsc.subcore_barrier()

        def stage_and_pack(row):
            pltpu.async_copy(
                n_hbm.at[pl.ds(row * ROW_LEN, ROW_LEN)], nrow_v, nsem
            ).wait()
            for c in range(NCHUNK):
                for j in range(CHUNK // L):
                    o = c * CHUNK + j * L
                    down = nrow_v[pl.ds(o, L)]
                    up = nrow_v[pl.ds(N_SITES + o, L)]
                    sp = nrow_v[pl.ds(2 * N_SITES + o, L)]
                    tok_v[c, pl.ds(j * L, L)] = (
                        up + 2 * down + 4 * ((sp + 1) >> 1)
                    )

        stage_and_pack(row0)
        for c in range(NCHUNK):
            pltpu.async_copy(emb_v.at[tok_v.at[c]], rows_v.at[c], gsem[c])

        def round_body(r, carry):
            row = row0 + r
            for c in range(NCHUNK):
                pltpu.make_async_copy(
                    emb_v.at[tok_v.at[c]], rows_v.at[c], gsem[c]
                ).wait()
                pltpu.async_copy(
                    rows_v.at[c],
                    out_hbm.at[pl.ds(row * N_SITES + c * CHUNK, CHUNK)],
                    wsem[c],
                )

            @pl.when(r + 1 < rows_per_w)
            def _():
                stage_and_pack(row + 1)
                for c in range(NCHUNK):
                    pltpu.make_async_copy(
                        rows_v.at[c],
                        out_hbm.at[pl.ds(row * N_SITES + c * CHUNK, CHUNK)],
                        wsem[c],
                    ).wait()
                    pltpu.async_copy(
                        emb_v.at[tok_v.at[c]], rows_v.at[c], gsem[c]
                    )

            return carry

        lax.fori_loop(0, rows_per_w, round_body, 0)

        last = row0 + rows_per_w - 1
        for c in range(NCHUNK):
            pltpu.make_async_copy(
                rows_v.at[c],
                out_hbm.at[pl.ds(last * N_SITES + c * CHUNK, CHUNK)],
                wsem[c],
            ).wait()

    return body


def kernel(n_flat, embedding):
    n = jnp.asarray(n_flat)
    if n.ndim == 1:
        n = n[None, :]
    batch = n.shape[0]
    body = _make_sc_kernel(batch)
    out = body(n.reshape(-1), embedding)
    return out.reshape(batch, N_SITES, D_MODEL)
